# baseline (device time: 184054 ns/iter reference)
import jax
import jax.numpy as jnp
from jax import lax
from jax.experimental import pallas as pl
from jax.experimental.pallas import tpu as pltpu

T = 2048
D = 4096
V_SHARD = 8192
VB = 512
NSTEP = V_SHARD // VB

VBH = 256
W_SCALE = 4.0
F8 = jnp.float8_e4m3fn


def _cast_body(x_ref, o_ref):
    o_ref[...] = (x_ref[...] * (1.0 / W_SCALE)).astype(F8)


def _cast_x(x):
    blk = 256
    return pl.pallas_call(
        _cast_body,
        grid=(T // blk,),
        in_specs=[pl.BlockSpec((blk, D), lambda i: (i, 0))],
        out_specs=pl.BlockSpec((blk, D), lambda i: (i, 0)),
        out_shape=jax.ShapeDtypeStruct((T, D), F8),
    )(x)


def _stats_body(x8_ref, w_ref, lbl_ref, out_ref, s_acc, ll_acc):
    j = pl.program_id(0)

    @pl.when(j == 0)
    def _():
        s_acc[...] = jnp.zeros_like(s_acc)
        ll_acc[...] = jnp.zeros_like(ll_acc)

    my_x = lax.axis_index("x")

    w8 = (w_ref[...] * W_SCALE).astype(F8)
    logits = jax.lax.dot(x8_ref[...], w8, preferred_element_type=jnp.float32)

    s_acc[...] += jnp.sum(jnp.exp(logits), axis=1, keepdims=True)

    lbl_local = lbl_ref[...] - my_x * V_SHARD - j * VB
    idx = jnp.clip(lbl_local, 0, VB - 1)
    grp = idx // 128
    lane = idx % 128
    sub = logits[:, 384:]
    for k in (2, 1, 0):
        sub = jnp.where(grp == k, logits[:, k * 128:(k + 1) * 128], sub)
    g = jnp.take_along_axis(sub, lane, axis=1)
    valid = (lbl_local >= 0) & (lbl_local < VB)
    ll_acc[...] += jnp.where(valid, g, 0.0)

    @pl.when(j == NSTEP - 1)
    def _():
        out_ref[:, 0:1] = s_acc[...]
        out_ref[:, 1:2] = ll_acc[...]


def _local_stats(x8, W, lbl2d):
    return pl.pallas_call(
        _stats_body,
        grid=(NSTEP,),
        in_specs=[
            pl.BlockSpec((T, D), lambda j: (0, 0)),
            pl.BlockSpec((D, VB), lambda j: (0, j)),
            pl.BlockSpec((T, 1), lambda j: (0, 0)),
        ],
        out_specs=pl.BlockSpec((T, 2), lambda j: (0, 0)),
        out_shape=jax.ShapeDtypeStruct((T, 2), jnp.float32),
        scratch_shapes=[
            pltpu.VMEM((T, 1), jnp.float32),
            pltpu.VMEM((T, 1), jnp.float32),
        ],
    )(x8, W, lbl2d)


def _exchange_body(stats_ref, out_ref, recv_buf, send_sem, recv_sem):
    my_x = lax.axis_index("x")
    my_y = lax.axis_index("y")
    my_z = lax.axis_index("z")
    peer = (1 - my_x, my_y, my_z)

    barrier = pltpu.get_barrier_semaphore()
    pl.semaphore_signal(
        barrier, inc=1, device_id=peer, device_id_type=pl.DeviceIdType.MESH
    )
    pl.semaphore_wait(barrier, 1)

    rdma = pltpu.make_async_remote_copy(
        src_ref=stats_ref,
        dst_ref=recv_buf,
        send_sem=send_sem,
        recv_sem=recv_sem,
        device_id=peer,
        device_id_type=pl.DeviceIdType.MESH,
    )
    rdma.start()
    rdma.wait()

    s_tot = stats_ref[0:1, :] + recv_buf[0:1, :]
    ll_tot = stats_ref[1:2, :] + recv_buf[1:2, :]
    out_ref[...] = jnp.log(s_tot) - ll_tot


def _exchange(stats_t):
    return pl.pallas_call(
        _exchange_body,
        in_specs=[pl.BlockSpec(memory_space=pltpu.VMEM)],
        out_specs=pl.BlockSpec(memory_space=pltpu.VMEM),
        out_shape=jax.ShapeDtypeStruct((1, T), jnp.float32),
        scratch_shapes=[
            pltpu.VMEM((2, T), jnp.float32),
            pltpu.SemaphoreType.DMA,
            pltpu.SemaphoreType.DMA,
        ],
        compiler_params=pltpu.CompilerParams(collective_id=0),
    )(stats_t)


def kernel(x, W, labels):
    x8 = _cast_x(x)
    stats = _local_stats(x8, W, labels.reshape(T, 1))
    nll = _exchange(stats.T)
    return nll.reshape(T)


# device time: 122528 ns/iter; 1.5021x vs baseline; 1.5021x over previous
import jax
import jax.numpy as jnp
from jax import lax
from jax.experimental import pallas as pl
from jax.experimental.pallas import tpu as pltpu

T = 2048
D = 4096
V_SHARD = 8192
VB = 512
NSTEP = V_SHARD // VB

VBH = 256
W_SCALE = 4.0
F8 = jnp.float8_e4m3fn


def _cast_body(x_ref, o_ref):
    o_ref[...] = (x_ref[...] * (1.0 / W_SCALE)).astype(F8)


def _cast_x(x):
    blk = 256
    return pl.pallas_call(
        _cast_body,
        grid=(T // blk,),
        in_specs=[pl.BlockSpec((blk, D), lambda i: (i, 0))],
        out_specs=pl.BlockSpec((blk, D), lambda i: (i, 0)),
        out_shape=jax.ShapeDtypeStruct((T, D), F8),
    )(x)


def _stats_body(x8_ref, w_ref, lbl_ref, out_ref, s_acc, ll_acc):
    j = pl.program_id(0)

    @pl.when(j == 0)
    def _():
        s_acc[...] = jnp.zeros_like(s_acc)
        ll_acc[...] = jnp.zeros_like(ll_acc)

    my_x = lax.axis_index("x")

    w8 = (w_ref[...] * W_SCALE).astype(F8)
    logits = jax.lax.dot(x8_ref[...], w8, preferred_element_type=jnp.float32)

    s_acc[...] += jnp.sum(jnp.exp(logits), axis=1, keepdims=True)

    lbl_local = lbl_ref[...] - my_x * V_SHARD - j * VB
    cols = lax.broadcasted_iota(jnp.int32, (T, VB), 1)
    ll_acc[...] += jnp.sum(
        jnp.where(cols == lbl_local, logits, 0.0), axis=1, keepdims=True
    )

    @pl.when(j == NSTEP - 1)
    def _():
        out_ref[:, 0:1] = s_acc[...]
        out_ref[:, 1:2] = ll_acc[...]


def _local_stats(x8, W, lbl2d):
    return pl.pallas_call(
        _stats_body,
        grid=(NSTEP,),
        in_specs=[
            pl.BlockSpec((T, D), lambda j: (0, 0)),
            pl.BlockSpec((D, VB), lambda j: (0, j)),
            pl.BlockSpec((T, 1), lambda j: (0, 0)),
        ],
        out_specs=pl.BlockSpec((T, 2), lambda j: (0, 0)),
        out_shape=jax.ShapeDtypeStruct((T, 2), jnp.float32),
        scratch_shapes=[
            pltpu.VMEM((T, 1), jnp.float32),
            pltpu.VMEM((T, 1), jnp.float32),
        ],
    )(x8, W, lbl2d)


def _exchange_body(stats_ref, out_ref, recv_buf, send_sem, recv_sem):
    my_x = lax.axis_index("x")
    my_y = lax.axis_index("y")
    my_z = lax.axis_index("z")
    peer = (1 - my_x, my_y, my_z)

    barrier = pltpu.get_barrier_semaphore()
    pl.semaphore_signal(
        barrier, inc=1, device_id=peer, device_id_type=pl.DeviceIdType.MESH
    )
    pl.semaphore_wait(barrier, 1)

    rdma = pltpu.make_async_remote_copy(
        src_ref=stats_ref,
        dst_ref=recv_buf,
        send_sem=send_sem,
        recv_sem=recv_sem,
        device_id=peer,
        device_id_type=pl.DeviceIdType.MESH,
    )
    rdma.start()
    rdma.wait()

    s_tot = stats_ref[0:1, :] + recv_buf[0:1, :]
    ll_tot = stats_ref[1:2, :] + recv_buf[1:2, :]
    out_ref[...] = jnp.log(s_tot) - ll_tot


def _exchange(stats_t):
    return pl.pallas_call(
        _exchange_body,
        in_specs=[pl.BlockSpec(memory_space=pltpu.VMEM)],
        out_specs=pl.BlockSpec(memory_space=pltpu.VMEM),
        out_shape=jax.ShapeDtypeStruct((1, T), jnp.float32),
        scratch_shapes=[
            pltpu.VMEM((2, T), jnp.float32),
            pltpu.SemaphoreType.DMA,
            pltpu.SemaphoreType.DMA,
        ],
        compiler_params=pltpu.CompilerParams(collective_id=0),
    )(stats_t)


def kernel(x, W, labels):
    x8 = _cast_x(x)
    stats = _local_stats(x8, W, labels.reshape(T, 1))
    nll = _exchange(stats.T)
    return nll.reshape(T)
